# baseline (speedup 1.0000x reference)
import functools
import numpy as np
import jax
import jax.numpy as jnp
from jax import lax
from jax.experimental import pallas as pl
from jax.experimental.pallas import tpu as pltpu
from jax.experimental.pallas import tpu_sc as plsc

D = 256
H = 8
L = 4
P = 4
DH = 32
B = 4
SPATIAL = ((64, 64), (32, 32), (16, 16), (8, 8))
NV = sum(h * w for h, w in SPATIAL)
NQ = NV
LVLBASE = (0, 4096, 5120, 5376)

QBLK = 680

_lane = np.arange(128)
_l_h = _lane // 16
_l_l = (_lane // 4) % 4
_l_p = _lane % 4
_WL = np.array([SPATIAL[l][1] for l in _l_l], np.float32).reshape(1, 128)
_HL = np.array([SPATIAL[l][0] for l in _l_l], np.float32).reshape(1, 128)
_WLI = _WL.astype(np.int32)
_BASE = (_l_h * NV + np.array([LVLBASE[l] for l in _l_l])).astype(np.int32).reshape(1, 128)
_PERMX = (_l_h * 32 + _l_l * 8 + _l_p * 2 + 0).astype(np.int32)
_PERMY = (_l_h * 32 + _l_l * 8 + _l_p * 2 + 1).astype(np.int32)
_S = (np.arange(4)[:, None] == _l_l[None, :]).astype(np.float32)
_G16 = (_lane[:, None] // 16 == _lane[None, :] // 16).astype(np.float32)


def _dot(a, b):
    return lax.dot_general(a, b, (((1,), (1,)), ((), ())),
                           preferred_element_type=jnp.float32)


def _vproj_kernel(value_ref, wv_ref, bv_ref, out_ref):
    v = _dot(value_ref[0], wv_ref[...]) + bv_ref[...]
    for h in range(H):
        out_ref[0, h] = v[:, h * DH:(h + 1) * DH]


def _qproj_kernel(q_ref, rx_ref, ry_ref, wsx_ref, bsx_ref, wsy_ref, bsy_ref,
                  wa_ref, ba_ref, s_ref, g16_ref, wl_ref, hl_ref, wli_ref,
                  base_ref, idx_ref, wts_ref):
    b = pl.program_id(0)
    q = q_ref[0]
    offx = _dot(q, wsx_ref[...]) + bsx_ref[...]
    offy = _dot(q, wsy_ref[...]) + bsy_ref[...]
    logits = _dot(q, wa_ref[...]) + ba_ref[...]
    m = jnp.max(logits, axis=1, keepdims=True)
    e = jnp.exp(logits - m)
    s = lax.dot_general(e, g16_ref[...], (((1,), (0,)), ((), ())),
                        preferred_element_type=jnp.float32)
    aw = e / s
    wl = wl_ref[...]
    hl = hl_ref[...]
    refx = lax.dot_general(rx_ref[0], s_ref[...], (((1,), (0,)), ((), ())),
                           preferred_element_type=jnp.float32)
    refy = lax.dot_general(ry_ref[0], s_ref[...], (((1,), (0,)), ((), ())),
                           preferred_element_type=jnp.float32)
    x = refx * wl + offx - 0.5
    y = refy * hl + offy - 0.5
    x0 = jnp.floor(x)
    y0 = jnp.floor(y)
    fx = x - x0
    fy = y - y0
    base = base_ref[...] + b * (H * NV)
    wli = wli_ref[...]
    for ty in (0, 1):
        for tx in (0, 1):
            xt = x0 + tx
            yt = y0 + ty
            valid = ((xt >= 0) & (xt <= wl - 1) & (yt >= 0) & (yt <= hl - 1))
            xc = jnp.clip(xt, 0.0, wl - 1)
            yc = jnp.clip(yt, 0.0, hl - 1)
            wx = fx if tx else 1.0 - fx
            wy = fy if ty else 1.0 - fy
            w = aw * wx * wy * valid.astype(jnp.float32)
            idx = base + yc.astype(jnp.int32) * wli + xc.astype(jnp.int32)
            t = ty * 2 + tx
            idx_ref[0, :, t * 128:(t + 1) * 128] = idx
            wts_ref[0, :, t * 128:(t + 1) * 128] = w


def _oproj_kernel(o_ref, wo_ref, bo_ref, id_ref, out_ref):
    out_ref[...] = _dot(o_ref[...], wo_ref[...]) + bo_ref[...] + id_ref[...]


NW = 32
QPW = B * NQ // NW
CQ = 2
NCHUNK = QPW // CQ


def _splat(vec, j):
    return lax.gather(
        vec, jnp.full((16, 1), j, jnp.int32),
        lax.GatherDimensionNumbers(offset_dims=(), collapsed_slice_dims=(0,),
                                   start_index_map=(0,)),
        (1,), mode=lax.GatherScatterMode.PROMISE_IN_BOUNDS)


def _sc_body(idx_hbm, wts_hbm, table_hbm, out_hbm, idx_v, wts_v, rows_v,
             out_v, sem_i0, sem_i1, sem_g0, sem_g1, sem_o0, sem_o1):
    cid = lax.axis_index("c")
    sid = lax.axis_index("s")
    wid = sid * 2 + cid
    sem_i = (sem_i0, sem_i1)
    sem_g = (sem_g0, sem_g1)
    sem_o = (sem_o0, sem_o1)

    def in_descs(c, p):
        qb = wid * QPW + c * CQ
        return (pltpu.make_async_copy(idx_hbm.at[pl.ds(qb, CQ)],
                                      idx_v.at[p], sem_i[p]),
                pltpu.make_async_copy(wts_hbm.at[pl.ds(qb * 512, CQ * 512)],
                                      wts_v.at[p], sem_i[p]))

    def issue_in(c, p):
        for d in in_descs(c, p):
            d.start()

    def wait_in(c, p):
        for d in in_descs(c, p):
            d.wait()

    def gather_descs(p):
        ds = []
        for q in range(CQ):
            for t in range(4):
                ds.append(pltpu.make_async_copy(
                    table_hbm.at[idx_v.at[p, q, t]],
                    rows_v.at[p, pl.ds((q * 4 + t) * 128, 128)], sem_g[p]))
        return ds

    def out_desc(c, p):
        qb = wid * QPW + c * CQ
        return pltpu.make_async_copy(out_v.at[p],
                                     out_hbm.at[pl.ds(qb, CQ)], sem_o[p])

    def compute(c, p):
        zero = jnp.zeros((16,), jnp.float32)
        for q in range(CQ):
            for h in range(H):
                w16s = [wts_v[p, pl.ds(q * 512 + t * 128 + h * 16, 16)]
                        for t in range(4)]
                def rbody(i4, acc, q=q, h=h, w16s=w16s):
                    a0, a1 = acc
                    for dj in range(4):
                        j = i4 * 4 + dj
                        for t in range(4):
                            r = q * 512 + t * 128 + h * 16 + j
                            wv = _splat(w16s[t], j)
                            a0 = a0 + wv * rows_v[p, r, 0:16]
                            a1 = a1 + wv * rows_v[p, r, 16:32]
                    return a0, a1
                a0, a1 = lax.fori_loop(0, 4, rbody, (zero, zero))
                out_v[p, q, pl.ds(h * DH, 16)] = a0
                out_v[p, q, pl.ds(h * DH + 16, 16)] = a1

    def phase(c, p):
        @pl.when(c + 1 < NCHUNK)
        def _():
            wait_in(c + 1, 1 - p)
            for d in gather_descs(1 - p):
                d.start()
        for d in gather_descs(p):
            d.wait()
        @pl.when(c >= 2)
        def _():
            out_desc(c - 2, p).wait()
        compute(c, p)
        out_desc(c, p).start()
        @pl.when(c + 2 < NCHUNK)
        def _():
            issue_in(c + 2, p)

    issue_in(0, 0)
    wait_in(0, 0)
    for d in gather_descs(0):
        d.start()
    issue_in(1, 1)

    def pair(i, carry):
        phase(2 * i, 0)
        phase(2 * i + 1, 1)
        return carry

    lax.fori_loop(0, NCHUNK // 2, pair, 0)
    out_desc(NCHUNK - 2, 0).wait()
    out_desc(NCHUNK - 1, 1).wait()


def kernel(query, value, reference_points, identity, spatial_shapes,
           Wv, bv, Ws, bs_, Wa, ba, Wo, bo):
    del spatial_shapes
    f32 = jnp.float32
    wsx = Ws[_PERMX]
    wsy = Ws[_PERMY]
    bsx = bs_[_PERMX].reshape(1, 128)
    bsy = bs_[_PERMY].reshape(1, 128)
    ba2 = ba.reshape(1, 128)
    bv2 = bv.reshape(1, D)
    bo2 = bo.reshape(1, D)

    table = pl.pallas_call(
        _vproj_kernel,
        grid=(B, NV // QBLK),
        in_specs=[
            pl.BlockSpec((1, QBLK, D), lambda b, i: (b, i, 0)),
            pl.BlockSpec((D, D), lambda b, i: (0, 0)),
            pl.BlockSpec((1, D), lambda b, i: (0, 0)),
        ],
        out_specs=pl.BlockSpec((1, H, QBLK, DH), lambda b, i: (b, 0, i, 0)),
        out_shape=jax.ShapeDtypeStruct((B, H, NV, DH), f32),
    )(value, Wv, bv2)

    refx = reference_points[..., 0]
    refy = reference_points[..., 1]
    idx, wts = pl.pallas_call(
        _qproj_kernel,
        grid=(B, NQ // QBLK),
        in_specs=[
            pl.BlockSpec((1, QBLK, D), lambda b, i: (b, i, 0)),
            pl.BlockSpec((1, QBLK, L), lambda b, i: (b, i, 0)),
            pl.BlockSpec((1, QBLK, L), lambda b, i: (b, i, 0)),
            pl.BlockSpec((128, D), lambda b, i: (0, 0)),
            pl.BlockSpec((1, 128), lambda b, i: (0, 0)),
            pl.BlockSpec((128, D), lambda b, i: (0, 0)),
            pl.BlockSpec((1, 128), lambda b, i: (0, 0)),
            pl.BlockSpec((128, D), lambda b, i: (0, 0)),
            pl.BlockSpec((1, 128), lambda b, i: (0, 0)),
            pl.BlockSpec((L, 128), lambda b, i: (0, 0)),
            pl.BlockSpec((128, 128), lambda b, i: (0, 0)),
            pl.BlockSpec((1, 128), lambda b, i: (0, 0)),
            pl.BlockSpec((1, 128), lambda b, i: (0, 0)),
            pl.BlockSpec((1, 128), lambda b, i: (0, 0)),
            pl.BlockSpec((1, 128), lambda b, i: (0, 0)),
        ],
        out_specs=[
            pl.BlockSpec((1, QBLK, 512), lambda b, i: (b, i, 0)),
            pl.BlockSpec((1, QBLK, 512), lambda b, i: (b, i, 0)),
        ],
        out_shape=[
            jax.ShapeDtypeStruct((B, NQ, 512), jnp.int32),
            jax.ShapeDtypeStruct((B, NQ, 512), f32),
        ],
    )(query, refx, refy, wsx, bsx, wsy, bsy, Wa, ba2,
      jnp.asarray(_S), jnp.asarray(_G16), jnp.asarray(_WL), jnp.asarray(_HL),
      jnp.asarray(_WLI), jnp.asarray(_BASE))

    sc = functools.partial(
        pl.kernel,
        out_type=jax.ShapeDtypeStruct((B * NQ, D), f32),
        mesh=plsc.VectorSubcoreMesh(core_axis_name="c", subcore_axis_name="s",
                                    num_cores=2, num_subcores=16),
        compiler_params=pltpu.CompilerParams(use_tc_tiling_on_sc=False),
        scratch_types=[
            pltpu.VMEM((2, CQ, 4, 128), jnp.int32),
            pltpu.VMEM((2, CQ * 512), f32),
            pltpu.VMEM((2, CQ * 512, DH), f32),
            pltpu.VMEM((2, CQ, D), f32),
            pltpu.SemaphoreType.DMA,
            pltpu.SemaphoreType.DMA,
            pltpu.SemaphoreType.DMA,
            pltpu.SemaphoreType.DMA,
            pltpu.SemaphoreType.DMA,
            pltpu.SemaphoreType.DMA,
        ],
    )(_sc_body)
    samp = sc(idx.reshape(B * NQ, 4, 128), wts.reshape(B * NQ * 512),
              table.reshape(B * H * NV, DH))

    out = pl.pallas_call(
        _oproj_kernel,
        grid=(B * NQ // QBLK,),
        in_specs=[
            pl.BlockSpec((QBLK, D), lambda i: (i, 0)),
            pl.BlockSpec((D, D), lambda i: (0, 0)),
            pl.BlockSpec((1, D), lambda i: (0, 0)),
            pl.BlockSpec((QBLK, D), lambda i: (i, 0)),
        ],
        out_specs=pl.BlockSpec((QBLK, D), lambda i: (i, 0)),
        out_shape=jax.ShapeDtypeStruct((B * NQ, D), f32),
    )(samp, Wo, bo2, identity.reshape(B * NQ, D))
    return out.reshape(B, NQ, D)


# speedup vs baseline: 1.2796x; 1.2796x over previous
import functools
import numpy as np
import jax
import jax.numpy as jnp
from jax import lax
from jax.experimental import pallas as pl
from jax.experimental.pallas import tpu as pltpu
from jax.experimental.pallas import tpu_sc as plsc

D = 256
H = 8
L = 4
P = 4
DH = 32
B = 4
SPATIAL = ((64, 64), (32, 32), (16, 16), (8, 8))
NV = sum(h * w for h, w in SPATIAL)
NQ = NV
LVLBASE = (0, 4096, 5120, 5376)

QBLK = 680

_lane = np.arange(128)
_l_h = _lane // 16
_l_l = (_lane // 4) % 4
_l_p = _lane % 4
_WL = np.array([SPATIAL[l][1] for l in _l_l], np.float32).reshape(1, 128)
_HL = np.array([SPATIAL[l][0] for l in _l_l], np.float32).reshape(1, 128)
_WLI = _WL.astype(np.int32)
_BASE = (_l_h * NV + np.array([LVLBASE[l] for l in _l_l])).astype(np.int32).reshape(1, 128)
_PERMX = (_l_h * 32 + _l_l * 8 + _l_p * 2 + 0).astype(np.int32)
_PERMY = (_l_h * 32 + _l_l * 8 + _l_p * 2 + 1).astype(np.int32)
_S = (np.arange(4)[:, None] == _l_l[None, :]).astype(np.float32)
_G16 = (_lane[:, None] // 16 == _lane[None, :] // 16).astype(np.float32)


def _dot(a, b):
    return lax.dot_general(a, b, (((1,), (1,)), ((), ())),
                           preferred_element_type=jnp.float32)


def _vproj_kernel(value_ref, wv_ref, bv_ref, out_ref):
    v = _dot(value_ref[0], wv_ref[...]) + bv_ref[...]
    for h in range(H):
        out_ref[0, h] = v[:, h * DH:(h + 1) * DH]


def _qproj_kernel(q_ref, rx_ref, ry_ref, wsx_ref, bsx_ref, wsy_ref, bsy_ref,
                  wa_ref, ba_ref, s_ref, g16_ref, wl_ref, hl_ref, wli_ref,
                  base_ref, idx_ref, wts_ref):
    b = pl.program_id(0)
    q = q_ref[0]
    offx = _dot(q, wsx_ref[...]) + bsx_ref[...]
    offy = _dot(q, wsy_ref[...]) + bsy_ref[...]
    logits = _dot(q, wa_ref[...]) + ba_ref[...]
    m = jnp.max(logits, axis=1, keepdims=True)
    e = jnp.exp(logits - m)
    s = lax.dot_general(e, g16_ref[...], (((1,), (0,)), ((), ())),
                        preferred_element_type=jnp.float32)
    aw = e / s
    wl = wl_ref[...]
    hl = hl_ref[...]
    refx = lax.dot_general(rx_ref[0], s_ref[...], (((1,), (0,)), ((), ())),
                           preferred_element_type=jnp.float32)
    refy = lax.dot_general(ry_ref[0], s_ref[...], (((1,), (0,)), ((), ())),
                           preferred_element_type=jnp.float32)
    x = refx * wl + offx - 0.5
    y = refy * hl + offy - 0.5
    x0 = jnp.floor(x)
    y0 = jnp.floor(y)
    fx = x - x0
    fy = y - y0
    base = base_ref[...] + b * (H * NV)
    wli = wli_ref[...]
    for ty in (0, 1):
        for tx in (0, 1):
            xt = x0 + tx
            yt = y0 + ty
            valid = ((xt >= 0) & (xt <= wl - 1) & (yt >= 0) & (yt <= hl - 1))
            xc = jnp.clip(xt, 0.0, wl - 1)
            yc = jnp.clip(yt, 0.0, hl - 1)
            wx = fx if tx else 1.0 - fx
            wy = fy if ty else 1.0 - fy
            w = aw * wx * wy * valid.astype(jnp.float32)
            idx = base + yc.astype(jnp.int32) * wli + xc.astype(jnp.int32)
            t = ty * 2 + tx
            idx_ref[0, :, t * 128:(t + 1) * 128] = idx
            wts_ref[0, :, t * 128:(t + 1) * 128] = w


def _oproj_kernel(o_ref, wo_ref, bo_ref, id_ref, out_ref):
    out_ref[...] = _dot(o_ref[...], wo_ref[...]) + bo_ref[...] + id_ref[...]


NW = 32
QPW = B * NQ // NW
CQ = 2
NCHUNK = QPW // CQ


def _splat(vec, j):
    return lax.gather(
        vec, jnp.full((16, 1), j, jnp.int32),
        lax.GatherDimensionNumbers(offset_dims=(), collapsed_slice_dims=(0,),
                                   start_index_map=(0,)),
        (1,), mode=lax.GatherScatterMode.PROMISE_IN_BOUNDS)


def _sc_body(idx_hbm, wts_hbm, table_hbm, out_hbm, idx_v, wts_v, rows_v,
             out_v, sem_i0, sem_i1, sem_g0, sem_g1, sem_o0, sem_o1):
    cid = lax.axis_index("c")
    sid = lax.axis_index("s")
    wid = sid * 2 + cid
    sem_i = (sem_i0, sem_i1)
    sem_g = (sem_g0, sem_g1)
    sem_o = (sem_o0, sem_o1)

    def in_descs(c, p):
        qb = wid * QPW + c * CQ
        return (pltpu.make_async_copy(idx_hbm.at[pl.ds(qb, CQ)],
                                      idx_v.at[p], sem_i[p]),
                pltpu.make_async_copy(wts_hbm.at[pl.ds(qb * 512, CQ * 512)],
                                      wts_v.at[p], sem_i[p]))

    def issue_in(c, p):
        for d in in_descs(c, p):
            d.start()

    def wait_in(c, p):
        for d in in_descs(c, p):
            d.wait()

    def gather_descs(p):
        ds = []
        for q in range(CQ):
            for t in range(4):
                ds.append(pltpu.make_async_copy(
                    table_hbm.at[idx_v.at[p, q, t]],
                    rows_v.at[p, pl.ds((q * 4 + t) * 128, 128)], sem_g[p]))
        return ds

    def out_desc(c, p):
        qb = wid * QPW + c * CQ
        return pltpu.make_async_copy(out_v.at[p],
                                     out_hbm.at[pl.ds(qb, CQ)], sem_o[p])

    def compute(c, p):
        zero = jnp.zeros((16,), jnp.float32)
        for q in range(CQ):
            for h in range(H):
                w16s = [wts_v[p, pl.ds(q * 512 + t * 128 + h * 16, 16)]
                        for t in range(4)]
                def rbody(j, acc, q=q, h=h, w16s=w16s):
                    a0, a1, b0, b1 = acc
                    for t in range(4):
                        r = q * 512 + t * 128 + h * 16 + j
                        wv = _splat(w16s[t], j)
                        if t % 2 == 0:
                            a0 = a0 + wv * rows_v[p, r, 0:16]
                            a1 = a1 + wv * rows_v[p, r, 16:32]
                        else:
                            b0 = b0 + wv * rows_v[p, r, 0:16]
                            b1 = b1 + wv * rows_v[p, r, 16:32]
                    return a0, a1, b0, b1
                a0, a1, b0, b1 = lax.fori_loop(
                    0, 16, rbody, (zero, zero, zero, zero))
                out_v[p, q, pl.ds(h * DH, 16)] = a0 + b0
                out_v[p, q, pl.ds(h * DH + 16, 16)] = a1 + b1

    def phase(c, p):
        @pl.when(c + 1 < NCHUNK)
        def _():
            wait_in(c + 1, 1 - p)
            for d in gather_descs(1 - p):
                d.start()
        for d in gather_descs(p):
            d.wait()
        @pl.when(c >= 2)
        def _():
            out_desc(c - 2, p).wait()
        compute(c, p)
        out_desc(c, p).start()
        @pl.when(c + 2 < NCHUNK)
        def _():
            issue_in(c + 2, p)

    issue_in(0, 0)
    wait_in(0, 0)
    for d in gather_descs(0):
        d.start()
    issue_in(1, 1)

    def pair(i, carry):
        phase(2 * i, 0)
        phase(2 * i + 1, 1)
        return carry

    lax.fori_loop(0, NCHUNK // 2, pair, 0)
    out_desc(NCHUNK - 2, 0).wait()
    out_desc(NCHUNK - 1, 1).wait()


def kernel(query, value, reference_points, identity, spatial_shapes,
           Wv, bv, Ws, bs_, Wa, ba, Wo, bo):
    del spatial_shapes
    f32 = jnp.float32
    wsx = Ws[_PERMX]
    wsy = Ws[_PERMY]
    bsx = bs_[_PERMX].reshape(1, 128)
    bsy = bs_[_PERMY].reshape(1, 128)
    ba2 = ba.reshape(1, 128)
    bv2 = bv.reshape(1, D)
    bo2 = bo.reshape(1, D)

    table = pl.pallas_call(
        _vproj_kernel,
        grid=(B, NV // QBLK),
        in_specs=[
            pl.BlockSpec((1, QBLK, D), lambda b, i: (b, i, 0)),
            pl.BlockSpec((D, D), lambda b, i: (0, 0)),
            pl.BlockSpec((1, D), lambda b, i: (0, 0)),
        ],
        out_specs=pl.BlockSpec((1, H, QBLK, DH), lambda b, i: (b, 0, i, 0)),
        out_shape=jax.ShapeDtypeStruct((B, H, NV, DH), f32),
    )(value, Wv, bv2)

    refx = reference_points[..., 0]
    refy = reference_points[..., 1]
    idx, wts = pl.pallas_call(
        _qproj_kernel,
        grid=(B, NQ // QBLK),
        in_specs=[
            pl.BlockSpec((1, QBLK, D), lambda b, i: (b, i, 0)),
            pl.BlockSpec((1, QBLK, L), lambda b, i: (b, i, 0)),
            pl.BlockSpec((1, QBLK, L), lambda b, i: (b, i, 0)),
            pl.BlockSpec((128, D), lambda b, i: (0, 0)),
            pl.BlockSpec((1, 128), lambda b, i: (0, 0)),
            pl.BlockSpec((128, D), lambda b, i: (0, 0)),
            pl.BlockSpec((1, 128), lambda b, i: (0, 0)),
            pl.BlockSpec((128, D), lambda b, i: (0, 0)),
            pl.BlockSpec((1, 128), lambda b, i: (0, 0)),
            pl.BlockSpec((L, 128), lambda b, i: (0, 0)),
            pl.BlockSpec((128, 128), lambda b, i: (0, 0)),
            pl.BlockSpec((1, 128), lambda b, i: (0, 0)),
            pl.BlockSpec((1, 128), lambda b, i: (0, 0)),
            pl.BlockSpec((1, 128), lambda b, i: (0, 0)),
            pl.BlockSpec((1, 128), lambda b, i: (0, 0)),
        ],
        out_specs=[
            pl.BlockSpec((1, QBLK, 512), lambda b, i: (b, i, 0)),
            pl.BlockSpec((1, QBLK, 512), lambda b, i: (b, i, 0)),
        ],
        out_shape=[
            jax.ShapeDtypeStruct((B, NQ, 512), jnp.int32),
            jax.ShapeDtypeStruct((B, NQ, 512), f32),
        ],
    )(query, refx, refy, wsx, bsx, wsy, bsy, Wa, ba2,
      jnp.asarray(_S), jnp.asarray(_G16), jnp.asarray(_WL), jnp.asarray(_HL),
      jnp.asarray(_WLI), jnp.asarray(_BASE))

    sc = functools.partial(
        pl.kernel,
        out_type=jax.ShapeDtypeStruct((B * NQ, D), f32),
        mesh=plsc.VectorSubcoreMesh(core_axis_name="c", subcore_axis_name="s",
                                    num_cores=2, num_subcores=16),
        compiler_params=pltpu.CompilerParams(use_tc_tiling_on_sc=False),
        scratch_types=[
            pltpu.VMEM((2, CQ, 4, 128), jnp.int32),
            pltpu.VMEM((2, CQ * 512), f32),
            pltpu.VMEM((2, CQ * 512, DH), f32),
            pltpu.VMEM((2, CQ, D), f32),
            pltpu.SemaphoreType.DMA,
            pltpu.SemaphoreType.DMA,
            pltpu.SemaphoreType.DMA,
            pltpu.SemaphoreType.DMA,
            pltpu.SemaphoreType.DMA,
            pltpu.SemaphoreType.DMA,
        ],
    )(_sc_body)
    samp = sc(idx.reshape(B * NQ, 4, 128), wts.reshape(B * NQ * 512),
              table.reshape(B * H * NV, DH))

    out = pl.pallas_call(
        _oproj_kernel,
        grid=(B * NQ // QBLK,),
        in_specs=[
            pl.BlockSpec((QBLK, D), lambda i: (i, 0)),
            pl.BlockSpec((D, D), lambda i: (0, 0)),
            pl.BlockSpec((1, D), lambda i: (0, 0)),
            pl.BlockSpec((QBLK, D), lambda i: (i, 0)),
        ],
        out_specs=pl.BlockSpec((QBLK, D), lambda i: (i, 0)),
        out_shape=jax.ShapeDtypeStruct((B * NQ, D), f32),
    )(samp, Wo, bo2, identity.reshape(B * NQ, D))
    return out.reshape(B, NQ, D)


# speedup vs baseline: 1.3177x; 1.0297x over previous
import functools
import numpy as np
import jax
import jax.numpy as jnp
from jax import lax
from jax.experimental import pallas as pl
from jax.experimental.pallas import tpu as pltpu
from jax.experimental.pallas import tpu_sc as plsc

D = 256
H = 8
L = 4
P = 4
DH = 32
B = 4
SPATIAL = ((64, 64), (32, 32), (16, 16), (8, 8))
NV = sum(h * w for h, w in SPATIAL)
NQ = NV
LVLBASE = (0, 4096, 5120, 5376)

QBLK = 680

_lane = np.arange(128)
_l_h = _lane // 16
_l_l = (_lane // 4) % 4
_l_p = _lane % 4
_WL = np.array([SPATIAL[l][1] for l in _l_l], np.float32).reshape(1, 128)
_HL = np.array([SPATIAL[l][0] for l in _l_l], np.float32).reshape(1, 128)
_WLI = _WL.astype(np.int32)
_BASE = (_l_h * NV + np.array([LVLBASE[l] for l in _l_l])).astype(np.int32).reshape(1, 128)
_PERMX = (_l_h * 32 + _l_l * 8 + _l_p * 2 + 0).astype(np.int32)
_PERMY = (_l_h * 32 + _l_l * 8 + _l_p * 2 + 1).astype(np.int32)
_S = (np.arange(4)[:, None] == _l_l[None, :]).astype(np.float32)
_G16 = (_lane[:, None] // 16 == _lane[None, :] // 16).astype(np.float32)


def _dot(a, b):
    return lax.dot_general(a, b, (((1,), (1,)), ((), ())),
                           preferred_element_type=jnp.float32)


def _vproj_kernel(value_ref, wv_ref, bv_ref, out_ref):
    v = _dot(value_ref[0], wv_ref[...]) + bv_ref[...]
    for h in range(H):
        out_ref[0, h] = v[:, h * DH:(h + 1) * DH]


def _qproj_kernel(q_ref, rx_ref, ry_ref, wsx_ref, bsx_ref, wsy_ref, bsy_ref,
                  wa_ref, ba_ref, s_ref, g16_ref, wl_ref, hl_ref, wli_ref,
                  base_ref, idx_ref, wts_ref):
    b = pl.program_id(0)
    q = q_ref[0]
    offx = _dot(q, wsx_ref[...]) + bsx_ref[...]
    offy = _dot(q, wsy_ref[...]) + bsy_ref[...]
    logits = _dot(q, wa_ref[...]) + ba_ref[...]
    m = jnp.max(logits, axis=1, keepdims=True)
    e = jnp.exp(logits - m)
    s = lax.dot_general(e, g16_ref[...], (((1,), (0,)), ((), ())),
                        preferred_element_type=jnp.float32)
    aw = e / s
    wl = wl_ref[...]
    hl = hl_ref[...]
    refx = lax.dot_general(rx_ref[0], s_ref[...], (((1,), (0,)), ((), ())),
                           preferred_element_type=jnp.float32)
    refy = lax.dot_general(ry_ref[0], s_ref[...], (((1,), (0,)), ((), ())),
                           preferred_element_type=jnp.float32)
    x = refx * wl + offx - 0.5
    y = refy * hl + offy - 0.5
    x0 = jnp.floor(x)
    y0 = jnp.floor(y)
    fx = x - x0
    fy = y - y0
    base = base_ref[...] + b * (H * NV)
    wli = wli_ref[...]
    for ty in (0, 1):
        for tx in (0, 1):
            xt = x0 + tx
            yt = y0 + ty
            valid = ((xt >= 0) & (xt <= wl - 1) & (yt >= 0) & (yt <= hl - 1))
            xc = jnp.clip(xt, 0.0, wl - 1)
            yc = jnp.clip(yt, 0.0, hl - 1)
            wx = fx if tx else 1.0 - fx
            wy = fy if ty else 1.0 - fy
            w = aw * wx * wy * valid.astype(jnp.float32)
            idx = base + yc.astype(jnp.int32) * wli + xc.astype(jnp.int32)
            t = ty * 2 + tx
            idx_ref[0, :, t * 128:(t + 1) * 128] = idx
            wts_ref[0, :, t * 128:(t + 1) * 128] = w


def _oproj_kernel(o_ref, wo_ref, bo_ref, id_ref, out_ref):
    out_ref[...] = _dot(o_ref[...], wo_ref[...]) + bo_ref[...] + id_ref[...]


NW = 32
QPW = B * NQ // NW
CQ = 2
NCHUNK = QPW // CQ


def _splat(vec, j):
    return lax.gather(
        vec, jnp.full((16, 1), j, jnp.int32),
        lax.GatherDimensionNumbers(offset_dims=(), collapsed_slice_dims=(0,),
                                   start_index_map=(0,)),
        (1,), mode=lax.GatherScatterMode.PROMISE_IN_BOUNDS)


def _sc_body(idx_hbm, wts_hbm, table_hbm, out_hbm, idx_v, wts_v, rows_v,
             out_v, sem_i0, sem_i1, sem_g0, sem_g1, sem_o0, sem_o1):
    cid = lax.axis_index("c")
    sid = lax.axis_index("s")
    wid = sid * 2 + cid
    sem_i = (sem_i0, sem_i1)
    sem_g = (sem_g0, sem_g1)
    sem_o = (sem_o0, sem_o1)

    def in_descs(c, p):
        qb = wid * QPW + c * CQ
        return (pltpu.make_async_copy(idx_hbm.at[pl.ds(qb, CQ)],
                                      idx_v.at[p], sem_i[p]),
                pltpu.make_async_copy(wts_hbm.at[pl.ds(qb * 512, CQ * 512)],
                                      wts_v.at[p], sem_i[p]))

    def issue_in(c, p):
        for d in in_descs(c, p):
            d.start()

    def wait_in(c, p):
        for d in in_descs(c, p):
            d.wait()

    def gather_descs(p):
        ds = []
        for q in range(CQ):
            for t in range(4):
                ds.append(pltpu.make_async_copy(
                    table_hbm.at[idx_v.at[p, q, t]],
                    rows_v.at[p, pl.ds((q * 4 + t) * 128, 128)], sem_g[p]))
        return ds

    def out_desc(c, p):
        qb = wid * QPW + c * CQ
        return pltpu.make_async_copy(out_v.at[p],
                                     out_hbm.at[pl.ds(qb, CQ)], sem_o[p])

    def compute(c, p):
        zero = jnp.zeros((16,), jnp.float32)
        for q in range(CQ):
            for h in range(H):
                w16s = [wts_v[p, pl.ds(q * 512 + t * 128 + h * 16, 16)]
                        for t in range(4)]
                def rbody(j, acc, q=q, h=h, w16s=w16s):
                    a0, a1 = acc
                    for t in range(4):
                        r = q * 512 + t * 128 + h * 16 + j
                        wv = _splat(w16s[t], j)
                        a0 = a0 + wv * rows_v[p, r, 0:16]
                        a1 = a1 + wv * rows_v[p, r, 16:32]
                    return a0, a1
                a0, a1 = lax.fori_loop(0, 16, rbody, (zero, zero))
                out_v[p, q, pl.ds(h * DH, 16)] = a0
                out_v[p, q, pl.ds(h * DH + 16, 16)] = a1

    def phase(c, p):
        @pl.when(c + 1 < NCHUNK)
        def _():
            wait_in(c + 1, 1 - p)
            for d in gather_descs(1 - p):
                d.start()
        for d in gather_descs(p):
            d.wait()
        @pl.when(c >= 2)
        def _():
            out_desc(c - 2, p).wait()
        compute(c, p)
        out_desc(c, p).start()
        @pl.when(c + 2 < NCHUNK)
        def _():
            issue_in(c + 2, p)

    issue_in(0, 0)
    wait_in(0, 0)
    for d in gather_descs(0):
        d.start()
    issue_in(1, 1)

    def pair(i, carry):
        phase(2 * i, 0)
        phase(2 * i + 1, 1)
        return carry

    lax.fori_loop(0, NCHUNK // 2, pair, 0)
    out_desc(NCHUNK - 2, 0).wait()
    out_desc(NCHUNK - 1, 1).wait()


def kernel(query, value, reference_points, identity, spatial_shapes,
           Wv, bv, Ws, bs_, Wa, ba, Wo, bo):
    del spatial_shapes
    f32 = jnp.float32
    wsx = Ws[_PERMX]
    wsy = Ws[_PERMY]
    bsx = bs_[_PERMX].reshape(1, 128)
    bsy = bs_[_PERMY].reshape(1, 128)
    ba2 = ba.reshape(1, 128)
    bv2 = bv.reshape(1, D)
    bo2 = bo.reshape(1, D)

    table = pl.pallas_call(
        _vproj_kernel,
        grid=(B, NV // QBLK),
        in_specs=[
            pl.BlockSpec((1, QBLK, D), lambda b, i: (b, i, 0)),
            pl.BlockSpec((D, D), lambda b, i: (0, 0)),
            pl.BlockSpec((1, D), lambda b, i: (0, 0)),
        ],
        out_specs=pl.BlockSpec((1, H, QBLK, DH), lambda b, i: (b, 0, i, 0)),
        out_shape=jax.ShapeDtypeStruct((B, H, NV, DH), f32),
    )(value, Wv, bv2)

    refx = reference_points[..., 0]
    refy = reference_points[..., 1]
    idx, wts = pl.pallas_call(
        _qproj_kernel,
        grid=(B, NQ // QBLK),
        in_specs=[
            pl.BlockSpec((1, QBLK, D), lambda b, i: (b, i, 0)),
            pl.BlockSpec((1, QBLK, L), lambda b, i: (b, i, 0)),
            pl.BlockSpec((1, QBLK, L), lambda b, i: (b, i, 0)),
            pl.BlockSpec((128, D), lambda b, i: (0, 0)),
            pl.BlockSpec((1, 128), lambda b, i: (0, 0)),
            pl.BlockSpec((128, D), lambda b, i: (0, 0)),
            pl.BlockSpec((1, 128), lambda b, i: (0, 0)),
            pl.BlockSpec((128, D), lambda b, i: (0, 0)),
            pl.BlockSpec((1, 128), lambda b, i: (0, 0)),
            pl.BlockSpec((L, 128), lambda b, i: (0, 0)),
            pl.BlockSpec((128, 128), lambda b, i: (0, 0)),
            pl.BlockSpec((1, 128), lambda b, i: (0, 0)),
            pl.BlockSpec((1, 128), lambda b, i: (0, 0)),
            pl.BlockSpec((1, 128), lambda b, i: (0, 0)),
            pl.BlockSpec((1, 128), lambda b, i: (0, 0)),
        ],
        out_specs=[
            pl.BlockSpec((1, QBLK, 512), lambda b, i: (b, i, 0)),
            pl.BlockSpec((1, QBLK, 512), lambda b, i: (b, i, 0)),
        ],
        out_shape=[
            jax.ShapeDtypeStruct((B, NQ, 512), jnp.int32),
            jax.ShapeDtypeStruct((B, NQ, 512), f32),
        ],
    )(query, refx, refy, wsx, bsx, wsy, bsy, Wa, ba2,
      jnp.asarray(_S), jnp.asarray(_G16), jnp.asarray(_WL), jnp.asarray(_HL),
      jnp.asarray(_WLI), jnp.asarray(_BASE))

    sc = functools.partial(
        pl.kernel,
        out_type=jax.ShapeDtypeStruct((B * NQ, D), f32),
        mesh=plsc.VectorSubcoreMesh(core_axis_name="c", subcore_axis_name="s",
                                    num_cores=2, num_subcores=16),
        compiler_params=pltpu.CompilerParams(use_tc_tiling_on_sc=False),
        scratch_types=[
            pltpu.VMEM((2, CQ, 4, 128), jnp.int32),
            pltpu.VMEM((2, CQ * 512), f32),
            pltpu.VMEM((2, CQ * 512, DH), f32),
            pltpu.VMEM((2, CQ, D), f32),
            pltpu.SemaphoreType.DMA,
            pltpu.SemaphoreType.DMA,
            pltpu.SemaphoreType.DMA,
            pltpu.SemaphoreType.DMA,
            pltpu.SemaphoreType.DMA,
            pltpu.SemaphoreType.DMA,
        ],
    )(_sc_body)
    samp = sc(idx.reshape(B * NQ, 4, 128), wts.reshape(B * NQ * 512),
              table.reshape(B * H * NV, DH))

    out = pl.pallas_call(
        _oproj_kernel,
        grid=(B * NQ // QBLK,),
        in_specs=[
            pl.BlockSpec((QBLK, D), lambda i: (i, 0)),
            pl.BlockSpec((D, D), lambda i: (0, 0)),
            pl.BlockSpec((1, D), lambda i: (0, 0)),
            pl.BlockSpec((QBLK, D), lambda i: (i, 0)),
        ],
        out_specs=pl.BlockSpec((QBLK, D), lambda i: (i, 0)),
        out_shape=jax.ShapeDtypeStruct((B * NQ, D), f32),
    )(samp, Wo, bo2, identity.reshape(B * NQ, D))
    return out.reshape(B, NQ, D)
